# baseline (speedup 1.0000x reference)
import functools

import jax
import jax.numpy as jnp
from jax import lax
from jax.experimental import pallas as pl
from jax.experimental.pallas import tpu as pltpu
from jax.experimental.pallas import tpu_sc as plsc

NC = 2
NS = 16
NW = NC * NS
LANES = 16
PDIM = 7
GB = 2


@functools.cache
def _make_sc_kernel(B, L, D):
    LW = L // NW
    DJ = D // LANES
    mesh = plsc.VectorSubcoreMesh(core_axis_name="c", subcore_axis_name="s")

    @functools.partial(
        pl.kernel,
        mesh=mesh,
        out_type=jax.ShapeDtypeStruct((B, L, D), jnp.float32),
        compiler_params=pltpu.CompilerParams(use_tc_tiling_on_sc=False),
        scratch_types=[
            pltpu.VMEM((B, LW), jnp.int32),
            pltpu.VMEM((B, LW, LANES), jnp.float32),
            pltpu.VMEM((LW, D), jnp.float32),
            pltpu.VMEM((PDIM, D), jnp.float32),
            pltpu.VMEM((D,), jnp.float32),
            pltpu.VMEM((2, LW, D), jnp.float32),
            pltpu.VMEM((2, LW, D), jnp.float32),
            pltpu.SemaphoreType.DMA,
            pltpu.SemaphoreType.DMA,
            pltpu.SemaphoreType.DMA,
            pltpu.SemaphoreType.DMA,
        ],
    )
    def k(ids_hbm, pros_hbm, table_hbm, pos_hbm, projw_hbm, projb_hbm,
          out_hbm, ids_v, pros_v, posb_v, w_v, b_v, rows_v, outb_v,
          sem_g0, sem_g1, sem_s0, sem_s1):
        wid = lax.axis_index("s") * NC + lax.axis_index("c")
        l0 = wid * LW
        sem_g = (sem_g0, sem_g1)
        sem_s = (sem_s0, sem_s1)

        pltpu.sync_copy(ids_hbm.at[wid], ids_v)
        pltpu.sync_copy(pros_hbm.at[wid], pros_v)
        pltpu.sync_copy(pos_hbm.at[pl.ds(l0, LW)], posb_v)
        pltpu.sync_copy(projw_hbm, w_v)
        pltpu.sync_copy(projb_hbm, b_v)

        def fold_i(i, c):
            for j in range(DJ):
                sl = pl.ds(j * LANES, LANES)
                posb_v[i, sl] = posb_v[i, sl] + b_v[sl]
            return c
        lax.fori_loop(0, LW, fold_i, None)

        def start_gather(p, b):
            return pltpu.async_copy(
                table_hbm.at[ids_v.at[b]], rows_v.at[p], sem_g[p])

        def start_scatter(p, b):
            return pltpu.async_copy(
                outb_v.at[p], out_hbm.at[b, pl.ds(l0, LW)], sem_s[p])

        def compute(p, b):
            def body_ib(ib, c):
                for g in range(GB):
                    i = ib * GB + g
                    pv = pros_v[b, i, :]
                    p0, p1, p2, p3 = pv[0], pv[1], pv[2], pv[3]
                    p4, p5, p6 = pv[4], pv[5], pv[6]
                    for j in range(DJ):
                        sl = pl.ds(j * LANES, LANES)
                        acc = rows_v[p, i, sl] + posb_v[i, sl]
                        acc = acc + p0 * w_v[0, sl]
                        acc = acc + p1 * w_v[1, sl]
                        acc = acc + p2 * w_v[2, sl]
                        acc = acc + p3 * w_v[3, sl]
                        acc = acc + p4 * w_v[4, sl]
                        acc = acc + p5 * w_v[5, sl]
                        acc = acc + p6 * w_v[6, sl]
                        outb_v[p, i, sl] = acc
                return c
            lax.fori_loop(0, LW // GB, body_ib, None)

        g0 = start_gather(0, 0)
        g1 = start_gather(1, 1)
        del g0, g1

        def body_t(t, c):
            for p in range(2):
                b = 2 * t + p
                pltpu.make_async_copy(
                    table_hbm.at[ids_v.at[b]], rows_v.at[p], sem_g[p]).wait()

                @pl.when(t > 0)
                def _():
                    pltpu.make_async_copy(
                        outb_v.at[p], out_hbm.at[b - 2, pl.ds(l0, LW)],
                        sem_s[p]).wait()

                compute(p, b)
                start_scatter(p, b)

                @pl.when(b + 2 < B)
                def _():
                    start_gather(p, b + 2)
            return c
        lax.fori_loop(0, B // 2, body_t, None)

        for p in range(2):
            b = B - 2 + p
            pltpu.make_async_copy(
                outb_v.at[p], out_hbm.at[b, pl.ds(l0, LW)], sem_s[p]).wait()

    return k


def kernel(token_ids, prosody_features, token_table, pos_table, proj_w,
           proj_b):
    B, L = token_ids.shape
    D = token_table.shape[1]
    LW = L // NW
    ids_prep = (token_ids.astype(jnp.int32)
                .reshape(B, NW, LW).transpose(1, 0, 2))
    pros_prep = jnp.pad(prosody_features, ((0, 0), (0, 0), (0, LANES - PDIM)))
    pros_prep = pros_prep.reshape(B, NW, LW, LANES).transpose(1, 0, 2, 3)
    k = _make_sc_kernel(B, L, D)
    return k(ids_prep, pros_prep, token_table, pos_table, proj_w, proj_b)


# speedup vs baseline: 3.2914x; 3.2914x over previous
import functools

import jax
import jax.numpy as jnp
from jax import lax
from jax.experimental import pallas as pl
from jax.experimental.pallas import tpu as pltpu
from jax.experimental.pallas import tpu_sc as plsc

NC = 2
NS = 16
NW = NC * NS
LANES = 16
PDIM = 7
GB = 2
UNROLL = 4


@functools.cache
def _make_sc_kernel(B, L, D):
    LW = L // NW
    DJ = D // LANES
    mesh = plsc.VectorSubcoreMesh(core_axis_name="c", subcore_axis_name="s")

    @functools.partial(
        pl.kernel,
        mesh=mesh,
        out_type=jax.ShapeDtypeStruct((B, L, D), jnp.float32),
        compiler_params=pltpu.CompilerParams(use_tc_tiling_on_sc=False),
        scratch_types=[
            pltpu.VMEM((B, LW), jnp.int32),
            pltpu.VMEM((2, LW, PDIM, LANES), jnp.float32),
            pltpu.VMEM((LW, D), jnp.float32),
            pltpu.VMEM((PDIM, D), jnp.float32),
            pltpu.VMEM((D,), jnp.float32),
            pltpu.VMEM((2, LW, D), jnp.float32),
            pltpu.VMEM((2, LW, D), jnp.float32),
            pltpu.SemaphoreType.DMA,
            pltpu.SemaphoreType.DMA,
            pltpu.SemaphoreType.DMA,
            pltpu.SemaphoreType.DMA,
        ],
    )
    def k(ids_hbm, pros_hbm, table_hbm, pos_hbm, projw_hbm, projb_hbm,
          out_hbm, ids_v, pb_v, posb_v, w_v, b_v, rows_v, outb_v,
          sem_g0, sem_g1, sem_s0, sem_s1):
        wid = lax.axis_index("s") * NC + lax.axis_index("c")
        l0 = wid * LW
        sem_g = (sem_g0, sem_g1)
        sem_s = (sem_s0, sem_s1)

        pltpu.sync_copy(ids_hbm.at[wid], ids_v)
        pltpu.sync_copy(pos_hbm.at[pl.ds(l0, LW)], posb_v)
        pltpu.sync_copy(projw_hbm, w_v)
        pltpu.sync_copy(projb_hbm, b_v)

        def fold_i(i, c):
            def fold_j(j, c2):
                sl = pl.ds(j * LANES, LANES)
                posb_v[i, sl] = posb_v[i, sl] + b_v[sl]
                return c2
            return lax.fori_loop(0, DJ, fold_j, c)
        lax.fori_loop(0, LW, fold_i, None)

        def start_gather(p, b):
            pltpu.async_copy(
                table_hbm.at[ids_v.at[b]], rows_v.at[p], sem_g[p])
            pltpu.sync_copy(pros_hbm.at[wid, b], pb_v.at[p])

        def wait_gather(p, b):
            pltpu.make_async_copy(
                table_hbm.at[ids_v.at[b]], rows_v.at[p], sem_g[p]).wait()

        def start_scatter(p, b):
            pltpu.async_copy(
                outb_v.at[p], out_hbm.at[b, pl.ds(l0, LW)], sem_s[p])

        def wait_scatter(p, b):
            pltpu.make_async_copy(
                outb_v.at[p], out_hbm.at[b, pl.ds(l0, LW)], sem_s[p]).wait()

        def compute(p):
            def body_ib(ib, c):
                i0 = ib * GB
                pvec = []
                for g in range(GB):
                    pvec.append(
                        [pb_v[p, i0 + g, kk, :] for kk in range(PDIM)])

                @plsc.parallel_loop(0, DJ, step=UNROLL)
                def body_j(j):
                    for u in range(UNROLL):
                        sl = pl.ds((j + u) * LANES, LANES)
                        w = [w_v[kk, sl] for kk in range(PDIM)]
                        for g in range(GB):
                            i = i0 + g
                            pk = pvec[g]
                            acc = rows_v[p, i, sl] + posb_v[i, sl]
                            m01 = pk[0] * w[0] + pk[1] * w[1]
                            m23 = pk[2] * w[2] + pk[3] * w[3]
                            m45 = pk[4] * w[4] + pk[5] * w[5]
                            acc = acc + (m01 + m23)
                            acc = acc + (m45 + pk[6] * w[6])
                            outb_v[p, i, sl] = acc
                return c
            lax.fori_loop(0, LW // GB, body_ib, None)

        start_gather(0, 0)
        start_gather(1, 1)

        def body_t(t, c):
            for p in range(2):
                b = 2 * t + p
                wait_gather(p, b)

                @pl.when(t > 0)
                def _():
                    wait_scatter(p, b - 2)

                compute(p)
                start_scatter(p, b)

                @pl.when(b + 2 < B)
                def _():
                    start_gather(p, b + 2)
            return c
        lax.fori_loop(0, B // 2, body_t, None)

        for p in range(2):
            wait_scatter(p, B - 2 + p)

    return k


def kernel(token_ids, prosody_features, token_table, pos_table, proj_w,
           proj_b):
    B, L = token_ids.shape
    D = token_table.shape[1]
    LW = L // NW
    ids_prep = (token_ids.astype(jnp.int32)
                .reshape(B, NW, LW).transpose(1, 0, 2))
    pros_prep = (prosody_features
                 .reshape(B, NW, LW, PDIM, 1).transpose(1, 0, 2, 3, 4))
    pros_prep = jnp.broadcast_to(
        pros_prep, (NW, B, LW, PDIM, LANES))
    k = _make_sc_kernel(B, L, D)
    return k(ids_prep, pros_prep, token_table, pos_table, proj_w, proj_b)


# speedup vs baseline: 7.5460x; 2.2927x over previous
import functools

import jax
import jax.numpy as jnp
from jax import lax
from jax.experimental import pallas as pl
from jax.experimental.pallas import tpu as pltpu
from jax.experimental.pallas import tpu_sc as plsc

NC = 2
NS = 16
NWA = 28
LANES = 16
PDIM = 7
LW = 16
GB = 2
UNROLL = 4


@functools.cache
def _make_sc_kernel(B, L, D):
    DJ = D // LANES
    mesh = plsc.VectorSubcoreMesh(core_axis_name="c", subcore_axis_name="s")

    @functools.partial(
        pl.kernel,
        mesh=mesh,
        out_type=jax.ShapeDtypeStruct((B, L, D), jnp.float32),
        scratch_types=[
            pltpu.VMEM((B, LW), jnp.int32),
            pltpu.VMEM((2, LW, LANES), jnp.float32),
            pltpu.VMEM((LW, D), jnp.float32),
            pltpu.VMEM((PDIM, D), jnp.float32),
            pltpu.VMEM((D,), jnp.float32),
            pltpu.VMEM((2, LW, D), jnp.float32),
            pltpu.VMEM((2, LW, D), jnp.float32),
            pltpu.SemaphoreType.DMA,
            pltpu.SemaphoreType.DMA,
            pltpu.SemaphoreType.DMA,
            pltpu.SemaphoreType.DMA,
            pltpu.SemaphoreType.DMA,
            pltpu.SemaphoreType.DMA,
        ],
    )
    def k(ids_hbm, pros_hbm, table_hbm, pos_hbm, projw_hbm, projb_hbm,
          out_hbm, ids_v, pb_v, posb_v, w_v, b_v, rows_v, outb_v,
          sem_g0, sem_g1, sem_s0, sem_s1, sem_p0, sem_p1):
        wid = lax.axis_index("s") * NC + lax.axis_index("c")

        @pl.when(wid < NWA)
        def _body():
            l0 = wid * LW
            sem_g = (sem_g0, sem_g1)
            sem_s = (sem_s0, sem_s1)
            sem_p = (sem_p0, sem_p1)

            pltpu.sync_copy(ids_hbm.at[wid], ids_v)
            pltpu.sync_copy(pos_hbm.at[wid], posb_v)
            pltpu.sync_copy(projw_hbm, w_v)
            pltpu.sync_copy(projb_hbm, b_v)

            def fold_i(i, c):
                def fold_j(j, c2):
                    sl = pl.ds(j * LANES, LANES)
                    posb_v[i, sl] = posb_v[i, sl] + b_v[sl]
                    return c2
                return lax.fori_loop(0, DJ, fold_j, c)
            lax.fori_loop(0, LW, fold_i, None)

            def start_fetch(p, b):
                pltpu.async_copy(
                    table_hbm.at[ids_v.at[b]], rows_v.at[p], sem_g[p])
                pltpu.async_copy(
                    pros_hbm.at[wid, b], pb_v.at[p], sem_p[p])

            def wait_fetch(p, b):
                pltpu.make_async_copy(
                    table_hbm.at[ids_v.at[b]], rows_v.at[p], sem_g[p]).wait()
                pltpu.make_async_copy(
                    pros_hbm.at[wid, b], pb_v.at[p], sem_p[p]).wait()

            def start_scatter(p, b):
                pltpu.async_copy(
                    outb_v.at[p], out_hbm.at[b, pl.ds(l0, LW)], sem_s[p])

            def wait_scatter(p, b):
                pltpu.make_async_copy(
                    outb_v.at[p], out_hbm.at[b, pl.ds(l0, LW)],
                    sem_s[p]).wait()

            def compute(p):
                def body_ib(ib, c):
                    i0 = ib * GB
                    pvec = []
                    for g in range(GB):
                        pv = pb_v[p, i0 + g, :]
                        pvec.append([pv[kk] for kk in range(PDIM)])

                    @plsc.parallel_loop(0, DJ, step=UNROLL)
                    def body_j(j):
                        for u in range(UNROLL):
                            sl = pl.ds((j + u) * LANES, LANES)
                            w = [w_v[kk, sl] for kk in range(PDIM)]
                            for g in range(GB):
                                i = i0 + g
                                pk = pvec[g]
                                acc = rows_v[p, i, sl] + posb_v[i, sl]
                                m01 = pk[0] * w[0] + pk[1] * w[1]
                                m23 = pk[2] * w[2] + pk[3] * w[3]
                                m45 = pk[4] * w[4] + pk[5] * w[5]
                                acc = acc + (m01 + m23)
                                acc = acc + (m45 + pk[6] * w[6])
                                outb_v[p, i, sl] = acc
                    return c
                lax.fori_loop(0, LW // GB, body_ib, None)

            start_fetch(0, 0)
            start_fetch(1, 1)

            def body_t(t, c):
                for p in range(2):
                    b = 2 * t + p
                    wait_fetch(p, b)

                    @pl.when(t > 0)
                    def _():
                        wait_scatter(p, b - 2)

                    compute(p)
                    start_scatter(p, b)

                    @pl.when(b + 2 < B)
                    def _():
                        start_fetch(p, b + 2)
                return c
            lax.fori_loop(0, B // 2, body_t, None)

            for p in range(2):
                wait_scatter(p, B - 2 + p)

    return k


def kernel(token_ids, prosody_features, token_table, pos_table, proj_w,
           proj_b):
    B, L = token_ids.shape
    D = token_table.shape[1]
    ids_prep = (token_ids.astype(jnp.int32)
                .reshape(B, NWA, LW).transpose(1, 0, 2))
    pros_prep = jnp.pad(prosody_features, ((0, 0), (0, 0), (0, LANES - PDIM)))
    pros_prep = (pros_prep.reshape(B, NWA, LW, LANES)
                 .transpose(1, 0, 2, 3))
    pos_prep = pos_table.reshape(NWA, LW, D)
    k = _make_sc_kernel(B, L, D)
    return k(ids_prep, pros_prep, token_table, pos_prep, proj_w, proj_b)


# speedup vs baseline: 8.6053x; 1.1404x over previous
import functools

import jax
import jax.numpy as jnp
from jax import lax
from jax.experimental import pallas as pl
from jax.experimental.pallas import tpu as pltpu
from jax.experimental.pallas import tpu_sc as plsc

NC = 2
NS = 16
NWA = 28
LANES = 16
PDIM = 7
LW = 16
GB = 4
UNROLL = 2


@functools.cache
def _make_sc_kernel(B, L, D):
    DJ = D // LANES
    mesh = plsc.VectorSubcoreMesh(core_axis_name="c", subcore_axis_name="s")

    @functools.partial(
        pl.kernel,
        mesh=mesh,
        out_type=jax.ShapeDtypeStruct((B, L, D), jnp.float32),
        scratch_types=[
            pltpu.VMEM((B, LW), jnp.int32),
            pltpu.VMEM((2, LW, LANES), jnp.float32),
            pltpu.VMEM((LW, D), jnp.float32),
            pltpu.VMEM((PDIM, D), jnp.float32),
            pltpu.VMEM((D,), jnp.float32),
            pltpu.VMEM((2, LW, D), jnp.float32),
            pltpu.VMEM((2, LW, D), jnp.float32),
            pltpu.SemaphoreType.DMA,
            pltpu.SemaphoreType.DMA,
            pltpu.SemaphoreType.DMA,
            pltpu.SemaphoreType.DMA,
            pltpu.SemaphoreType.DMA,
            pltpu.SemaphoreType.DMA,
        ],
    )
    def k(ids_hbm, pros_hbm, table_hbm, pos_hbm, projw_hbm, projb_hbm,
          out_hbm, ids_v, pb_v, posb_v, w_v, b_v, rows_v, outb_v,
          sem_g0, sem_g1, sem_s0, sem_s1, sem_p0, sem_p1):
        wid = lax.axis_index("s") * NC + lax.axis_index("c")

        @pl.when(wid < NWA)
        def _body():
            l0 = wid * LW
            sem_g = (sem_g0, sem_g1)
            sem_s = (sem_s0, sem_s1)
            sem_p = (sem_p0, sem_p1)

            pltpu.sync_copy(ids_hbm.at[wid], ids_v)
            pltpu.sync_copy(pos_hbm.at[wid], posb_v)
            pltpu.sync_copy(projw_hbm, w_v)
            pltpu.sync_copy(projb_hbm, b_v)

            def fold_i(i, c):
                def fold_j(j, c2):
                    sl = pl.ds(j * LANES, LANES)
                    posb_v[i, sl] = posb_v[i, sl] + b_v[sl]
                    return c2
                return lax.fori_loop(0, DJ, fold_j, c)
            lax.fori_loop(0, LW, fold_i, None)

            def start_fetch(p, b):
                pltpu.async_copy(
                    table_hbm.at[ids_v.at[b]], rows_v.at[p], sem_g[p])
                pltpu.async_copy(
                    pros_hbm.at[wid, b], pb_v.at[p], sem_p[p])

            def wait_fetch(p, b):
                pltpu.make_async_copy(
                    table_hbm.at[ids_v.at[b]], rows_v.at[p], sem_g[p]).wait()
                pltpu.make_async_copy(
                    pros_hbm.at[wid, b], pb_v.at[p], sem_p[p]).wait()

            def start_scatter(p, b):
                pltpu.async_copy(
                    outb_v.at[p], out_hbm.at[b, pl.ds(l0, LW)], sem_s[p])

            def wait_scatter(p, b):
                pltpu.make_async_copy(
                    outb_v.at[p], out_hbm.at[b, pl.ds(l0, LW)],
                    sem_s[p]).wait()

            def compute(p):
                def body_ib(ib, c):
                    i0 = ib * GB
                    pvec = []
                    for g in range(GB):
                        pv = pb_v[p, i0 + g, :]
                        pvec.append([pv[kk] for kk in range(PDIM)])

                    @plsc.parallel_loop(0, DJ, step=UNROLL)
                    def body_j(j):
                        for u in range(UNROLL):
                            sl = pl.ds((j + u) * LANES, LANES)
                            w = [w_v[kk, sl] for kk in range(PDIM)]
                            for g in range(GB):
                                i = i0 + g
                                pk = pvec[g]
                                acc = rows_v[p, i, sl] + posb_v[i, sl]
                                m01 = pk[0] * w[0] + pk[1] * w[1]
                                m23 = pk[2] * w[2] + pk[3] * w[3]
                                m45 = pk[4] * w[4] + pk[5] * w[5]
                                acc = acc + (m01 + m23)
                                acc = acc + (m45 + pk[6] * w[6])
                                outb_v[p, i, sl] = acc
                    return c
                lax.fori_loop(0, LW // GB, body_ib, None)

            start_fetch(0, 0)
            start_fetch(1, 1)

            def body_t(t, c):
                for p in range(2):
                    b = 2 * t + p
                    wait_fetch(p, b)

                    @pl.when(t > 0)
                    def _():
                        wait_scatter(p, b - 2)

                    compute(p)
                    start_scatter(p, b)

                    @pl.when(b + 2 < B)
                    def _():
                        start_fetch(p, b + 2)
                return c
            lax.fori_loop(0, B // 2, body_t, None)

            for p in range(2):
                wait_scatter(p, B - 2 + p)

    return k


def kernel(token_ids, prosody_features, token_table, pos_table, proj_w,
           proj_b):
    B, L = token_ids.shape
    D = token_table.shape[1]
    ids_prep = (token_ids.astype(jnp.int32)
                .reshape(B, NWA, LW).transpose(1, 0, 2))
    pros_prep = jnp.pad(prosody_features, ((0, 0), (0, 0), (0, LANES - PDIM)))
    pros_prep = (pros_prep.reshape(B, NWA, LW, LANES)
                 .transpose(1, 0, 2, 3))
    pos_prep = pos_table.reshape(NWA, LW, D)
    k = _make_sc_kernel(B, L, D)
    return k(ids_prep, pros_prep, token_table, pos_prep, proj_w, proj_b)
